# Initial kernel scaffold; baseline (speedup 1.0000x reference)
import jax
import jax.numpy as jnp
from jax.experimental import pallas as pl


def kernel(feats, labels, predict, cluster_center):
    raise NotImplementedError("write your pallas kernel here")


# speedup vs baseline: 34.2551x; 34.2551x over previous
import functools

import jax
import jax.numpy as jnp
from jax import lax
from jax.experimental import pallas as pl
from jax.experimental.pallas import tpu as pltpu

NUM_CLASSES = 19
K = 40
DIM = 64
TEMP = 0.1
BASE_TEMP = 2.0
LAMB = 25.0
MU = 0.9999
IGNORE = 255
SINKHORN_ITERS = 25
CK = NUM_CLASSES * K
CKP = 768

BLK = 1024


def _k1_body(cc_ref, x_ref, yhat_ref, y_ref,
             pt_ref, phi_ref, cls_ref, w_ref, xn_ref):
    x = x_ref[...]
    nrm = jnp.sqrt(jnp.sum(x * x, axis=1, keepdims=True))
    xn = x / jnp.maximum(nrm, 1e-12)
    xn_ref[...] = xn

    yhat = yhat_ref[...]
    y = y_ref[...]
    valid = (yhat != IGNORE) & (y != IGNORE)
    cls = jnp.where(valid, yhat, 0).astype(jnp.int32)
    cls_ref[...] = cls
    w_ref[...] = valid.astype(jnp.float32)

    cc = cc_ref[...]
    sT = lax.dot_general(cc, xn, (((1,), (1,)), ((), ())),
                         preferred_element_type=jnp.float32)

    row = lax.broadcasted_iota(jnp.int32, (CKP, BLK), 0)
    live = row < CK
    sm = jnp.where(live, sT, -1e30)
    m = jnp.max(sm, axis=0, keepdims=True)
    e = jnp.where(live, jnp.exp((sm - m) * (1.0 / TEMP)), 0.0)
    lse = m * (1.0 / TEMP) + jnp.log(jnp.sum(e, axis=0, keepdims=True))

    acc = jnp.zeros((K, BLK), jnp.float32)
    for c in range(NUM_CLASSES):
        mc = (cls == c).astype(jnp.float32)
        acc = acc + mc * lax.slice(sT, (c * K, 0), (c * K + K, BLK))
    mk = jnp.max(acc, axis=0, keepdims=True)
    ek = jnp.exp(acc - mk)
    ps = ek / jnp.sum(ek, axis=0, keepdims=True)
    lp = LAMB * jnp.log(ps + 1e-12)
    lp = lp - jnp.max(lp, axis=0, keepdims=True)
    pt_ref[...] = jnp.exp(lp)
    phi_ref[...] = acc * (1.0 / TEMP) - lse


def _k2_body(pt_ref, phi_ref, cls_ref, w_ref,
             sub_ref, cnt_ref, loss_ref):
    M = pt_ref.shape[1]
    P = pt_ref[...]
    cls = cls_ref[...]
    w = w_ref[...]

    oh = (lax.broadcasted_iota(jnp.int32, (NUM_CLASSES, M), 0)
          == cls).astype(jnp.float32)
    counts = jnp.sum(oh, axis=1, keepdims=True)
    inv_n = 1.0 / jnp.maximum(counts, 1.0)
    inv_n_row = jnp.sum(oh * inv_n, axis=0, keepdims=True)
    inv_k = jnp.float32(1.0 / K)

    def it(_, carry):
        c, _rT = carry
        ohc = oh * c
        colT = lax.dot_general(P, ohc, (((1,), (1,)), ((), ())),
                               preferred_element_type=jnp.float32)
        rT = inv_k / jnp.maximum(colT, 1e-30)
        rg = lax.dot_general(rT, oh, (((1,), (0,)), ((), ())),
                             preferred_element_type=jnp.float32)
        rows = jnp.sum(P * rg, axis=0, keepdims=True)
        c2 = inv_n_row / jnp.maximum(rows, 1e-30)
        return (c2, rT)

    c0 = inv_n_row
    rT0 = jnp.full((K, NUM_CLASSES), inv_k, jnp.float32)
    c, rT = lax.fori_loop(0, SINKHORN_ITERS, it, (c0, rT0))

    rg = lax.dot_general(rT, oh, (((1,), (0,)), ((), ())),
                         preferred_element_type=jnp.float32)
    a = P * c * rg
    ma = jnp.max(a, axis=0, keepdims=True)
    i40 = lax.broadcasted_iota(jnp.int32, (K, M), 0)
    L = jnp.min(jnp.where(a == ma, i40, K), axis=0, keepdims=True)
    sub_ref[...] = cls * K + L

    ohk = (i40 == L).astype(jnp.float32)
    cnt_ref[...] = lax.dot_general(oh, ohk, (((1,), (1,)), ((), ())),
                                   preferred_element_type=jnp.float32)

    pos = jnp.sum(ohk * phi_ref[...], axis=0, keepdims=True)
    num = jnp.sum(pos * w, axis=1, keepdims=True)
    wsum = jnp.sum(w, axis=1, keepdims=True)
    loss_ref[...] = -(TEMP / BASE_TEMP) * num / jnp.maximum(wsum, 1.0)


def _k4_body(s0_ref, s1_ref, cnt_ref, cc_ref, out_ref):
    s = s0_ref[...] + s1_ref[...]
    nc = s / jnp.maximum(cnt_ref[...], 1.0)
    up = MU * cc_ref[...] + (1.0 - MU) * nc
    nrm = jnp.sqrt(jnp.sum(up * up, axis=1, keepdims=True))
    out_ref[...] = up / jnp.maximum(nrm, 1e-12)


def _make_sc_scatter(M):
    from jax.experimental.pallas import tpu_sc as plsc

    info = plsc.get_sparse_core_info()
    nc, ns = info.num_cores, info.num_subcores
    nw = nc * ns
    rows_per_tile = M // nw
    CH = 256
    n_chunks = rows_per_tile // CH
    JB = CH // 128
    IDXR = rows_per_tile // 128

    mesh = plsc.VectorSubcoreMesh(core_axis_name="c", subcore_axis_name="s")

    @functools.partial(
        pl.kernel, mesh=mesh,
        out_type=jax.ShapeDtypeStruct((nc, CK, DIM), jnp.float32),
        scratch_types=[
            pltpu.VMEM((CH, DIM), jnp.float32),
            pltpu.VMEM((IDXR, 128), jnp.int32),
            pltpu.VMEM_SHARED((CK, DIM), jnp.float32),
        ],
    )
    def sc_scatter(xn_hbm, sub_hbm, zeros_hbm, out_hbm, data_v, idx_v, shared):
        cid = lax.axis_index("c")
        sid = lax.axis_index("s")
        wid = cid * ns + sid

        @pl.when(sid == 0)
        def _():
            pltpu.sync_copy(zeros_hbm, shared)

        base0 = pl.multiple_of(wid * rows_per_tile, rows_per_tile)
        pltpu.sync_copy(sub_hbm.at[pl.ds(pl.multiple_of(base0 // 128, IDXR),
                                         IDXR)], idx_v)
        plsc.subcore_barrier()

        for t in range(n_chunks):
            base = pl.multiple_of(wid * rows_per_tile + t * CH, CH)
            pltpu.sync_copy(xn_hbm.at[pl.ds(base, CH)], data_v)
            for j in range(JB):
                pltpu.sync_copy(data_v.at[pl.ds(j * 128, 128)],
                                shared.at[idx_v.at[t * JB + j]], add=True)

        plsc.subcore_barrier()

        @pl.when(sid == 0)
        def _():
            pltpu.sync_copy(shared, out_hbm.at[cid])

    return sc_scatter


def kernel(feats, labels, predict, cluster_center):
    B, N, D = feats.shape
    M = B * N
    x = feats.reshape(M, D)
    yhat = predict.reshape(1, M).astype(jnp.int32)
    y = labels.reshape(1, M).astype(jnp.int32)
    cc_flat = cluster_center.reshape(CK, D)
    cc_pad = jnp.pad(cc_flat, ((0, CKP - CK), (0, 0)))

    nblk = M // BLK
    pt, phi, clsr, wr, xn = pl.pallas_call(
        _k1_body,
        grid=(nblk,),
        in_specs=[
            pl.BlockSpec((CKP, D), lambda i: (0, 0)),
            pl.BlockSpec((BLK, D), lambda i: (i, 0)),
            pl.BlockSpec((1, BLK), lambda i: (0, i)),
            pl.BlockSpec((1, BLK), lambda i: (0, i)),
        ],
        out_specs=[
            pl.BlockSpec((K, BLK), lambda i: (0, i)),
            pl.BlockSpec((K, BLK), lambda i: (0, i)),
            pl.BlockSpec((1, BLK), lambda i: (0, i)),
            pl.BlockSpec((1, BLK), lambda i: (0, i)),
            pl.BlockSpec((BLK, D), lambda i: (i, 0)),
        ],
        out_shape=[
            jax.ShapeDtypeStruct((K, M), jnp.float32),
            jax.ShapeDtypeStruct((K, M), jnp.float32),
            jax.ShapeDtypeStruct((1, M), jnp.int32),
            jax.ShapeDtypeStruct((1, M), jnp.float32),
            jax.ShapeDtypeStruct((M, D), jnp.float32),
        ],
    )(cc_pad, x, yhat, y)

    sub_r, cnts, loss = pl.pallas_call(
        _k2_body,
        grid=(1,),
        in_specs=[
            pl.BlockSpec((K, M), lambda i: (0, 0)),
            pl.BlockSpec((K, M), lambda i: (0, 0)),
            pl.BlockSpec((1, M), lambda i: (0, 0)),
            pl.BlockSpec((1, M), lambda i: (0, 0)),
        ],
        out_specs=[
            pl.BlockSpec((1, M), lambda i: (0, 0)),
            pl.BlockSpec((NUM_CLASSES, K), lambda i: (0, 0)),
            pl.BlockSpec((1, 1), lambda i: (0, 0)),
        ],
        out_shape=[
            jax.ShapeDtypeStruct((1, M), jnp.int32),
            jax.ShapeDtypeStruct((NUM_CLASSES, K), jnp.float32),
            jax.ShapeDtypeStruct((1, 1), jnp.float32),
        ],
    )(pt, phi, clsr, wr)

    sub2d = sub_r.reshape(M // 128, 128)
    zeros = jnp.zeros((CK, DIM), jnp.float32)
    sums2 = _make_sc_scatter(M)(xn, sub2d, zeros)

    cnt_col = cnts.reshape(CK, 1)
    updated = pl.pallas_call(
        _k4_body,
        grid=(1,),
        in_specs=[
            pl.BlockSpec((CK, D), lambda i: (0, 0)),
            pl.BlockSpec((CK, D), lambda i: (0, 0)),
            pl.BlockSpec((CK, 1), lambda i: (0, 0)),
            pl.BlockSpec((CK, D), lambda i: (0, 0)),
        ],
        out_specs=pl.BlockSpec((CK, D), lambda i: (0, 0)),
        out_shape=jax.ShapeDtypeStruct((CK, D), jnp.float32),
    )(sums2[0], sums2[1], cnt_col, cc_flat)

    return loss[0, 0], updated.reshape(NUM_CLASSES, K, DIM)
